# baseline (speedup 1.0000x reference)
import functools

import jax
import jax.numpy as jnp
import numpy as np
from jax import lax
from jax.experimental import pallas as pl
from jax.experimental.pallas import tpu as pltpu
from jax.experimental.pallas import tpu_sc as plsc

_N = 10000
_D = 128
_HALF = 64
_W32 = 32
_E = 320000
_NITER = 5

_NC = 2
_NS = 16

_NP = 10112
_RT = _NP // _NS
_FR = 79
_NFIN = _RT // _FR

_CHUNK = 128
_CPT = 160
_EPT = _CPT * _CHUNK
_EPAD = _EPT * _NS

_DEGW = 16

_H = 512
_HID = 256
_OUT = 64
_BR = 400
_GRID = _N // _BR

_PERM = np.concatenate(
    [32 * l + np.concatenate([np.arange(16) * 2, np.arange(16) * 2 + 1])
     for l in range(2)]).astype(np.int32)


def _sc_body(x_hbm, xw_hbm, edges_hbm, hs_hbm, hsb_hbm,
             packed_v, sidx_v, didx_v, rows_w, rows_f, ones_v, degloc_v,
             facc_v, fh_v, fbw_v, acc_sh, deg_sh, gsems, ssems):
    c = lax.axis_index("c")
    s = lax.axis_index("s")
    row_base = s * _RT

    pltpu.sync_copy(edges_hbm.at[s], packed_v)

    zero16 = jnp.zeros((16,), jnp.float32)
    ones16 = jnp.full((16,), 1.0, jnp.float32)
    mask14 = jnp.full((16,), 0x3FFF, jnp.int32)
    mhi = jnp.full((16,), -65536, jnp.int32)
    rnd = jnp.full((16,), 0x8000, jnp.int32)

    def _unpack(j, slot):
        for l in range(_CHUNK // 16):
            sl = pl.ds(l * 16, 16)
            p = packed_v[j, sl]
            sidx_v[slot, sl] = p & mask14
            didx_v[slot, sl] = lax.shift_right_logical(p, 14)

    def _fill_ones(i, _):
        ones_v[i, :] = ones16
        return 0
    lax.fori_loop(0, _CHUNK, _fill_ones, 0)

    def _zero_deg(i, _):
        degloc_v[i, :] = zero16
        return 0
    lax.fori_loop(0, _RT, _zero_deg, 0)
    pltpu.sync_copy(degloc_v, deg_sh.at[pl.ds(row_base, _RT), :])

    def _zero_fh(i, _):
        for c4 in range(4):
            fh_v[i, pl.ds(c4 * 16, 16)] = zero16
        return 0

    def _zero_acc_slice():
        lax.fori_loop(0, _FR, _zero_fh, 0)
        for i in range(_NFIN):
            pltpu.sync_copy(fh_v, acc_sh.at[pl.ds(row_base + i * _FR, _FR), :])

    for k in range(_NITER):
        tblw = xw_hbm.at[c] if k == 0 else hsb_hbm.at[k - 1, c]
        tblf = x_hbm.at[c] if k == 0 else hs_hbm.at[k - 1, c]

        _zero_acc_slice()
        plsc.subcore_barrier()

        def _gfire(g):
            bank = g % 4
            _unpack(g, bank)
            pltpu.async_copy(tblw.at[sidx_v.at[bank]],
                             rows_w.at[bank], gsems.at[bank])

        def _gwait(g):
            bank = g % 4
            pltpu.make_async_copy(tblw.at[sidx_v.at[bank]],
                                  rows_w.at[bank], gsems.at[bank]).wait()

        def _convert(g):
            gb = g % 4
            fb = g % 2

            def _crow(r8, _):
                for u in range(8):
                    r = r8 * 8 + u
                    for l in range(2):
                        w = rows_w[gb, r, pl.ds(l * 16, 16)]
                        e = lax.bitcast_convert_type(lax.shift_left(w, 16),
                                                     jnp.float32)
                        o = lax.bitcast_convert_type(w & mhi, jnp.float32)
                        rows_f[fb, r, pl.ds(l * 32, 16)] = e
                        rows_f[fb, r, pl.ds(l * 32 + 16, 16)] = o
                return 0
            lax.fori_loop(0, _CHUNK // 8, _crow, 0)

        def _sfire(g):
            fb = g % 2
            bank = g % 4
            pltpu.async_copy(rows_f.at[fb],
                             acc_sh.at[didx_v.at[bank]], ssems.at[fb],
                             add=True)
            if k == 0:
                pltpu.async_copy(ones_v, deg_sh.at[didx_v.at[bank]],
                                 ssems.at[fb], add=True)

        def _sdrain(g):
            fb = g % 2
            bank = g % 4
            pltpu.make_async_copy(rows_f.at[fb],
                                  acc_sh.at[didx_v.at[bank]],
                                  ssems.at[fb]).wait()
            if k == 0:
                pltpu.make_async_copy(ones_v, deg_sh.at[didx_v.at[bank]],
                                      ssems.at[fb]).wait()

        _gfire(0)
        _gfire(1)

        def _egroup(g, _):
            @pl.when(g >= 2)
            def _drain_old():
                _sdrain(g - 2)

            @pl.when(g + 2 < _CPT)
            def _prefetch():
                _gfire(g + 2)

            _gwait(g)
            _convert(g)
            _sfire(g)
            return 0
        lax.fori_loop(0, _CPT, _egroup, 0)
        _sdrain(_CPT - 2)
        _sdrain(_CPT - 1)
        plsc.subcore_barrier()

        if k == 0:
            pltpu.sync_copy(deg_sh.at[pl.ds(row_base, _RT), :], degloc_v)

            def _invd(r, _):
                degloc_v[r, :] = 1.0 / (degloc_v[r, :] + 1.0)
                return 0
            lax.fori_loop(0, _RT, _invd, 0)

        def _fin(i, _):
            rows = pl.ds(row_base + i * _FR, _FR)
            pltpu.sync_copy(acc_sh.at[rows, :], facc_v)
            pltpu.sync_copy(tblf.at[rows, :], fh_v)

            def _rowbody(r, _2):
                iv = degloc_v[i * _FR + r, :]
                for c4 in range(4):
                    sl = pl.ds(c4 * 16, 16)
                    facc_v[r, sl] = (facc_v[r, sl] + fh_v[r, sl]) * iv
                for l in range(2):
                    e = lax.bitcast_convert_type(
                        facc_v[r, pl.ds(l * 32, 16)], jnp.int32)
                    o = lax.bitcast_convert_type(
                        facc_v[r, pl.ds(l * 32 + 16, 16)], jnp.int32)
                    ew = lax.shift_right_logical(e + rnd, 16)
                    ow = (o + rnd) & mhi
                    fbw_v[r, pl.ds(l * 16, 16)] = ew | ow
                return 0
            lax.fori_loop(0, _FR, _rowbody, 0)
            pltpu.sync_copy(facc_v, hs_hbm.at[k, c, rows, :])
            pltpu.sync_copy(fbw_v, hsb_hbm.at[k, c, rows, :])
            return 0
        lax.fori_loop(0, _NFIN, _fin, 0)


@functools.cache
def _make_sc_powers():
    return pl.kernel(
        _sc_body,
        out_type=(
            jax.ShapeDtypeStruct((_NITER, _NC, _NP, _HALF), jnp.float32),
            jax.ShapeDtypeStruct((_NITER, _NC, _NP, _W32), jnp.int32),
        ),
        mesh=plsc.VectorSubcoreMesh(core_axis_name="c", subcore_axis_name="s"),
        compiler_params=pltpu.CompilerParams(use_tc_tiling_on_sc=False),
        scratch_types=[
            pltpu.VMEM((_CPT, _CHUNK), jnp.int32),
            pltpu.VMEM((4, _CHUNK), jnp.int32),
            pltpu.VMEM((4, _CHUNK), jnp.int32),
            pltpu.VMEM((4, _CHUNK, _W32), jnp.int32),
            pltpu.VMEM((2, _CHUNK, _HALF), jnp.float32),
            pltpu.VMEM((_CHUNK, _DEGW), jnp.float32),
            pltpu.VMEM((_RT, _DEGW), jnp.float32),
            pltpu.VMEM((_FR, _HALF), jnp.float32),
            pltpu.VMEM((_FR, _HALF), jnp.float32),
            pltpu.VMEM((_FR, _W32), jnp.int32),
            pltpu.VMEM_SHARED((_NP, _HALF), jnp.float32),
            pltpu.VMEM_SHARED((_NP, _DEGW), jnp.float32),
            pltpu.SemaphoreType.DMA((4,)),
            pltpu.SemaphoreType.DMA((2,)),
        ],
    )


def _mlp_body(hs_ref, Wc_ref, bc_ref, W1_ref, b1_ref, W2_ref, b2_ref, o_ref):
    def mm(k, c, w_ref, r0):
        return jnp.dot(hs_ref[k, c], w_ref[pl.ds(r0, _HALF), :],
                       preferred_element_type=jnp.float32)

    acc2 = bc_ref[...].astype(jnp.float32)
    for idx in range(8):
        k, c = 1 + idx // 2, idx % 2
        acc2 = acc2 + mm(k, c, Wc_ref, idx * _HALF)
    out2 = jnp.maximum(acc2, 0.0)

    accm = b1_ref[...].astype(jnp.float32)
    for idx in range(8):
        k, c = idx // 2, idx % 2
        accm = accm + mm(k, c, W1_ref, idx * _HALF)
    accm = accm + jnp.dot(out2, W1_ref[pl.ds(_H, _H), :],
                          preferred_element_type=jnp.float32)
    hm = jnp.maximum(accm, 0.0)

    o_ref[...] = jnp.dot(hm, W2_ref[...],
                         preferred_element_type=jnp.float32) + b2_ref[...]


@functools.partial(jax.jit, static_argnames=())
def _mlp_head(hs, W_conv, b_conv, W1, b1, W2, b2):
    return pl.pallas_call(
        _mlp_body,
        grid=(_GRID,),
        in_specs=[
            pl.BlockSpec((_NITER, _NC, _BR, _HALF), lambda i: (0, 0, i, 0)),
            pl.BlockSpec((_H, _H), lambda i: (0, 0)),
            pl.BlockSpec((1, _H), lambda i: (0, 0)),
            pl.BlockSpec((2 * _H, _HID), lambda i: (0, 0)),
            pl.BlockSpec((1, _HID), lambda i: (0, 0)),
            pl.BlockSpec((_HID, _OUT), lambda i: (0, 0)),
            pl.BlockSpec((1, _OUT), lambda i: (0, 0)),
        ],
        out_specs=pl.BlockSpec((_BR, _OUT), lambda i: (i, 0)),
        out_shape=jax.ShapeDtypeStruct((_N, _OUT), jnp.float32),
    )(hs, W_conv, b_conv, W1, b1, W2, b2)


def kernel(x, edge_index, W_conv, b_conv, W1, b1, W2, b2):
    x2 = x.reshape(_N, _NC, _HALF).transpose(1, 0, 2)
    x_p = jnp.concatenate(
        [x2, jnp.zeros((_NC, _NP - _N, _HALF), jnp.float32)], axis=1)

    x_pi = x_p[:, :, _PERM]
    xw = jax.lax.bitcast_convert_type(
        x_p.astype(jnp.bfloat16).reshape(_NC, _NP, _W32, 2), jnp.int32)

    src = edge_index[0].astype(jnp.int32)
    dst = edge_index[1].astype(jnp.int32)
    packed = src | (dst << 14)
    pad = jnp.full((_EPAD - _E,), _N | (_N << 14), jnp.int32)
    edges = jnp.concatenate([packed, pad]).reshape(_NS, _CPT, _CHUNK)

    hs, _ = _make_sc_powers()(x_pi, xw, edges)

    rows_p = np.concatenate([b * _HALF + _PERM for b in range(8)])
    Wc_p = W_conv[rows_p, :]
    W1_p = jnp.concatenate([W1[:_H][rows_p, :], W1[_H:]], axis=0)

    return _mlp_head(hs, Wc_p, b_conv.reshape(1, _H), W1_p,
                     b1.reshape(1, _HID), W2, b2.reshape(1, _OUT))


# speedup vs baseline: 1.3373x; 1.3373x over previous
import functools

import jax
import jax.numpy as jnp
import numpy as np
from jax import lax
from jax.experimental import pallas as pl
from jax.experimental.pallas import tpu as pltpu
from jax.experimental.pallas import tpu_sc as plsc

_N = 10000
_D = 128
_HALF = 64
_W32 = 32
_E = 320000
_NITER = 5

_NC = 2
_NS = 16

_NP = 10112
_RT = _NP // _NS
_FR = 79
_NFIN = _RT // _FR

_CHUNK = 128
_CPT = 160
_EPT = _CPT * _CHUNK
_EPAD = _EPT * _NS

_DEGW = 16

_H = 512
_HID = 256
_OUT = 64
_BR = 400
_GRID = _N // _BR

_PERM = np.concatenate(
    [32 * l + np.concatenate([np.arange(16) * 2, np.arange(16) * 2 + 1])
     for l in range(2)]).astype(np.int32)


def _sc_body(x_hbm, xw_hbm, edges_hbm, hs_hbm, hsb_hbm,
             packed_v, sidx_v, didx_v, rows_w, rows_f, ones_v, degloc_v,
             facc_v, fh_v, fbw_v, acc_sh, deg_sh, gsems, ssems):
    c = lax.axis_index("c")
    s = lax.axis_index("s")
    row_base = s * _RT

    pltpu.sync_copy(edges_hbm.at[s], packed_v)

    zero16 = jnp.zeros((16,), jnp.float32)
    ones16 = jnp.full((16,), 1.0, jnp.float32)
    mask14 = jnp.full((16,), 0x3FFF, jnp.int32)
    mhi = jnp.full((16,), -65536, jnp.int32)
    rnd = jnp.full((16,), 0x8000, jnp.int32)

    def _unpack(j, slot):
        for l in range(_CHUNK // 16):
            sl = pl.ds(l * 16, 16)
            p = packed_v[j, sl]
            sidx_v[slot, sl] = p & mask14
            didx_v[slot, sl] = lax.shift_right_logical(p, 14)

    def _fill_ones(i, _):
        ones_v[i, :] = ones16
        return 0
    lax.fori_loop(0, _CHUNK, _fill_ones, 0)

    def _zero_deg(i, _):
        degloc_v[i, :] = zero16
        return 0
    lax.fori_loop(0, _RT, _zero_deg, 0)
    pltpu.sync_copy(degloc_v, deg_sh.at[pl.ds(row_base, _RT), :])

    def _zero_fh(i, _):
        for c4 in range(4):
            fh_v[i, pl.ds(c4 * 16, 16)] = zero16
        return 0

    def _zero_acc_slice():
        lax.fori_loop(0, _FR, _zero_fh, 0)
        for i in range(_NFIN):
            pltpu.sync_copy(fh_v, acc_sh.at[pl.ds(row_base + i * _FR, _FR), :])

    for k in range(_NITER):
        tblw = xw_hbm.at[c] if k == 0 else hsb_hbm.at[k - 1, c]
        tblf = x_hbm.at[c] if k == 0 else hs_hbm.at[k - 1, c]

        _zero_acc_slice()
        plsc.subcore_barrier()

        def _gfire(g):
            bank = g % 4
            _unpack(g, bank)
            pltpu.async_copy(tblw.at[sidx_v.at[bank]],
                             rows_w.at[bank], gsems.at[bank])

        def _gwait(g):
            bank = g % 4
            pltpu.make_async_copy(tblw.at[sidx_v.at[bank]],
                                  rows_w.at[bank], gsems.at[bank]).wait()

        def _convert(g):
            gb = g % 4
            fb = g % 2

            @plsc.parallel_loop(0, _CHUNK, 1, unroll=8)
            def _crow(r):
                for l in range(2):
                    w = rows_w[gb, r, pl.ds(l * 16, 16)]
                    e = lax.bitcast_convert_type(lax.shift_left(w, 16),
                                                 jnp.float32)
                    o = lax.bitcast_convert_type(w & mhi, jnp.float32)
                    rows_f[fb, r, pl.ds(l * 32, 16)] = e
                    rows_f[fb, r, pl.ds(l * 32 + 16, 16)] = o

        def _sfire(g):
            fb = g % 2
            bank = g % 4
            pltpu.async_copy(rows_f.at[fb],
                             acc_sh.at[didx_v.at[bank]], ssems.at[fb],
                             add=True)
            if k == 0:
                pltpu.async_copy(ones_v, deg_sh.at[didx_v.at[bank]],
                                 ssems.at[fb], add=True)

        def _sdrain(g):
            fb = g % 2
            bank = g % 4
            pltpu.make_async_copy(rows_f.at[fb],
                                  acc_sh.at[didx_v.at[bank]],
                                  ssems.at[fb]).wait()
            if k == 0:
                pltpu.make_async_copy(ones_v, deg_sh.at[didx_v.at[bank]],
                                      ssems.at[fb]).wait()

        _gfire(0)
        _gfire(1)

        def _egroup(g, _):
            @pl.when(g >= 2)
            def _drain_old():
                _sdrain(g - 2)

            @pl.when(g + 2 < _CPT)
            def _prefetch():
                _gfire(g + 2)

            _gwait(g)
            _convert(g)
            _sfire(g)
            return 0
        lax.fori_loop(0, _CPT, _egroup, 0)
        _sdrain(_CPT - 2)
        _sdrain(_CPT - 1)
        plsc.subcore_barrier()

        if k == 0:
            pltpu.sync_copy(deg_sh.at[pl.ds(row_base, _RT), :], degloc_v)

            def _invd(r, _):
                degloc_v[r, :] = 1.0 / (degloc_v[r, :] + 1.0)
                return 0
            lax.fori_loop(0, _RT, _invd, 0)

        def _fin(i, _):
            rows = pl.ds(row_base + i * _FR, _FR)
            pltpu.sync_copy(acc_sh.at[rows, :], facc_v)
            pltpu.sync_copy(tblf.at[rows, :], fh_v)

            def _rowbody(r, _2):
                iv = degloc_v[i * _FR + r, :]
                for c4 in range(4):
                    sl = pl.ds(c4 * 16, 16)
                    facc_v[r, sl] = (facc_v[r, sl] + fh_v[r, sl]) * iv
                for l in range(2):
                    e = lax.bitcast_convert_type(
                        facc_v[r, pl.ds(l * 32, 16)], jnp.int32)
                    o = lax.bitcast_convert_type(
                        facc_v[r, pl.ds(l * 32 + 16, 16)], jnp.int32)
                    ew = lax.shift_right_logical(e + rnd, 16)
                    ow = (o + rnd) & mhi
                    fbw_v[r, pl.ds(l * 16, 16)] = ew | ow
                return 0
            lax.fori_loop(0, _FR, _rowbody, 0)
            pltpu.sync_copy(facc_v, hs_hbm.at[k, c, rows, :])
            pltpu.sync_copy(fbw_v, hsb_hbm.at[k, c, rows, :])
            return 0
        lax.fori_loop(0, _NFIN, _fin, 0)


@functools.cache
def _make_sc_powers():
    return pl.kernel(
        _sc_body,
        out_type=(
            jax.ShapeDtypeStruct((_NITER, _NC, _NP, _HALF), jnp.float32),
            jax.ShapeDtypeStruct((_NITER, _NC, _NP, _W32), jnp.int32),
        ),
        mesh=plsc.VectorSubcoreMesh(core_axis_name="c", subcore_axis_name="s"),
        compiler_params=pltpu.CompilerParams(use_tc_tiling_on_sc=False),
        scratch_types=[
            pltpu.VMEM((_CPT, _CHUNK), jnp.int32),
            pltpu.VMEM((4, _CHUNK), jnp.int32),
            pltpu.VMEM((4, _CHUNK), jnp.int32),
            pltpu.VMEM((4, _CHUNK, _W32), jnp.int32),
            pltpu.VMEM((2, _CHUNK, _HALF), jnp.float32),
            pltpu.VMEM((_CHUNK, _DEGW), jnp.float32),
            pltpu.VMEM((_RT, _DEGW), jnp.float32),
            pltpu.VMEM((_FR, _HALF), jnp.float32),
            pltpu.VMEM((_FR, _HALF), jnp.float32),
            pltpu.VMEM((_FR, _W32), jnp.int32),
            pltpu.VMEM_SHARED((_NP, _HALF), jnp.float32),
            pltpu.VMEM_SHARED((_NP, _DEGW), jnp.float32),
            pltpu.SemaphoreType.DMA((4,)),
            pltpu.SemaphoreType.DMA((2,)),
        ],
    )


def _mlp_body(hs_ref, Wc_ref, bc_ref, W1_ref, b1_ref, W2_ref, b2_ref, o_ref):
    def mm(k, c, w_ref, r0):
        return jnp.dot(hs_ref[k, c], w_ref[pl.ds(r0, _HALF), :],
                       preferred_element_type=jnp.float32)

    acc2 = bc_ref[...].astype(jnp.float32)
    for idx in range(8):
        k, c = 1 + idx // 2, idx % 2
        acc2 = acc2 + mm(k, c, Wc_ref, idx * _HALF)
    out2 = jnp.maximum(acc2, 0.0)

    accm = b1_ref[...].astype(jnp.float32)
    for idx in range(8):
        k, c = idx // 2, idx % 2
        accm = accm + mm(k, c, W1_ref, idx * _HALF)
    accm = accm + jnp.dot(out2, W1_ref[pl.ds(_H, _H), :],
                          preferred_element_type=jnp.float32)
    hm = jnp.maximum(accm, 0.0)

    o_ref[...] = jnp.dot(hm, W2_ref[...],
                         preferred_element_type=jnp.float32) + b2_ref[...]


@functools.partial(jax.jit, static_argnames=())
def _mlp_head(hs, W_conv, b_conv, W1, b1, W2, b2):
    return pl.pallas_call(
        _mlp_body,
        grid=(_GRID,),
        in_specs=[
            pl.BlockSpec((_NITER, _NC, _BR, _HALF), lambda i: (0, 0, i, 0)),
            pl.BlockSpec((_H, _H), lambda i: (0, 0)),
            pl.BlockSpec((1, _H), lambda i: (0, 0)),
            pl.BlockSpec((2 * _H, _HID), lambda i: (0, 0)),
            pl.BlockSpec((1, _HID), lambda i: (0, 0)),
            pl.BlockSpec((_HID, _OUT), lambda i: (0, 0)),
            pl.BlockSpec((1, _OUT), lambda i: (0, 0)),
        ],
        out_specs=pl.BlockSpec((_BR, _OUT), lambda i: (i, 0)),
        out_shape=jax.ShapeDtypeStruct((_N, _OUT), jnp.float32),
    )(hs, W_conv, b_conv, W1, b1, W2, b2)


def kernel(x, edge_index, W_conv, b_conv, W1, b1, W2, b2):
    x2 = x.reshape(_N, _NC, _HALF).transpose(1, 0, 2)
    x_p = jnp.concatenate(
        [x2, jnp.zeros((_NC, _NP - _N, _HALF), jnp.float32)], axis=1)

    x_pi = x_p[:, :, _PERM]
    xw = jax.lax.bitcast_convert_type(
        x_p.astype(jnp.bfloat16).reshape(_NC, _NP, _W32, 2), jnp.int32)

    src = edge_index[0].astype(jnp.int32)
    dst = edge_index[1].astype(jnp.int32)
    packed = src | (dst << 14)
    pad = jnp.full((_EPAD - _E,), _N | (_N << 14), jnp.int32)
    edges = jnp.concatenate([packed, pad]).reshape(_NS, _CPT, _CHUNK)

    hs, _ = _make_sc_powers()(x_pi, xw, edges)

    rows_p = np.concatenate([b * _HALF + _PERM for b in range(8)])
    Wc_p = W_conv[rows_p, :]
    W1_p = jnp.concatenate([W1[:_H][rows_p, :], W1[_H:]], axis=0)

    return _mlp_head(hs, Wc_p, b_conv.reshape(1, _H), W1_p,
                     b1.reshape(1, _HID), W2, b2.reshape(1, _OUT))


# speedup vs baseline: 1.3784x; 1.0307x over previous
import functools

import jax
import jax.numpy as jnp
import numpy as np
from jax import lax
from jax.experimental import pallas as pl
from jax.experimental.pallas import tpu as pltpu
from jax.experimental.pallas import tpu_sc as plsc

_N = 10000
_D = 128
_HALF = 64
_W32 = 32
_E = 320000
_NITER = 5

_NC = 2
_NS = 16

_NP = 10112
_RT = _NP // _NS
_FR = 79
_NFIN = _RT // _FR

_CHUNK = 128
_CPT = 160
_EPT = _CPT * _CHUNK
_EPAD = _EPT * _NS

_DEGW = 16

_H = 512
_HID = 256
_OUT = 64
_BR = 400
_GRID = _N // _BR

_PERM = np.concatenate(
    [32 * l + np.concatenate([np.arange(16) * 2, np.arange(16) * 2 + 1])
     for l in range(2)]).astype(np.int32)


def _sc_body(x_hbm, xw_hbm, edges_hbm, hs_hbm, hsb_hbm,
             packed_v, sidx_v, didx_v, rows_w, rows_f, ones_v, degloc_v,
             facc_v, fh_v, fbw_v, acc_sh, deg_sh, gsems, ssems):
    c = lax.axis_index("c")
    s = lax.axis_index("s")
    row_base = s * _RT

    pltpu.sync_copy(edges_hbm.at[s], packed_v)

    zero16 = jnp.zeros((16,), jnp.float32)
    ones16 = jnp.full((16,), 1.0, jnp.float32)
    mask14 = jnp.full((16,), 0x3FFF, jnp.int32)
    mhi = jnp.full((16,), -65536, jnp.int32)
    rnd = jnp.full((16,), 0x8000, jnp.int32)

    def _unpack(j, slot):
        for l in range(_CHUNK // 16):
            sl = pl.ds(l * 16, 16)
            p = packed_v[j, sl]
            sidx_v[slot, sl] = p & mask14
            didx_v[slot, sl] = lax.shift_right_logical(p, 14)

    @plsc.parallel_loop(0, _CHUNK, 1, unroll=8)
    def _fill_ones(i):
        ones_v[i, :] = ones16

    @plsc.parallel_loop(0, _RT, 1, unroll=8)
    def _zero_deg(i):
        degloc_v[i, :] = zero16
    pltpu.sync_copy(degloc_v, deg_sh.at[pl.ds(row_base, _RT), :])

    def _zero_acc_slice():
        @plsc.parallel_loop(0, _FR, 1, unroll=8)
        def _zero_fh(i):
            for c4 in range(4):
                fh_v[i, pl.ds(c4 * 16, 16)] = zero16
        for i in range(_NFIN):
            pltpu.sync_copy(fh_v, acc_sh.at[pl.ds(row_base + i * _FR, _FR), :])

    for k in range(_NITER):
        tblw = xw_hbm.at[c] if k == 0 else hsb_hbm.at[k - 1, c]
        tblf = x_hbm.at[c] if k == 0 else hs_hbm.at[k - 1, c]

        _zero_acc_slice()
        plsc.subcore_barrier()

        def _gfire(g):
            bank = g % 4
            _unpack(g, bank)
            pltpu.async_copy(tblw.at[sidx_v.at[bank]],
                             rows_w.at[bank], gsems.at[bank])

        def _gwait(g):
            bank = g % 4
            pltpu.make_async_copy(tblw.at[sidx_v.at[bank]],
                                  rows_w.at[bank], gsems.at[bank]).wait()

        def _convert(g):
            gb = g % 4
            fb = g % 2

            @plsc.parallel_loop(0, _CHUNK, 1, unroll=8)
            def _crow(r):
                for l in range(2):
                    w = rows_w[gb, r, pl.ds(l * 16, 16)]
                    e = lax.bitcast_convert_type(lax.shift_left(w, 16),
                                                 jnp.float32)
                    o = lax.bitcast_convert_type(w & mhi, jnp.float32)
                    rows_f[fb, r, pl.ds(l * 32, 16)] = e
                    rows_f[fb, r, pl.ds(l * 32 + 16, 16)] = o

        def _sfire(g):
            fb = g % 2
            bank = g % 4
            pltpu.async_copy(rows_f.at[fb],
                             acc_sh.at[didx_v.at[bank]], ssems.at[fb],
                             add=True)
            if k == 0:
                pltpu.async_copy(ones_v, deg_sh.at[didx_v.at[bank]],
                                 ssems.at[fb], add=True)

        def _sdrain(g):
            fb = g % 2
            bank = g % 4
            pltpu.make_async_copy(rows_f.at[fb],
                                  acc_sh.at[didx_v.at[bank]],
                                  ssems.at[fb]).wait()
            if k == 0:
                pltpu.make_async_copy(ones_v, deg_sh.at[didx_v.at[bank]],
                                      ssems.at[fb]).wait()

        _gfire(0)
        _gfire(1)

        def _egroup(g, _):
            @pl.when(g >= 2)
            def _drain_old():
                _sdrain(g - 2)

            @pl.when(g + 2 < _CPT)
            def _prefetch():
                _gfire(g + 2)

            _gwait(g)
            _convert(g)
            _sfire(g)
            return 0
        lax.fori_loop(0, _CPT, _egroup, 0)
        _sdrain(_CPT - 2)
        _sdrain(_CPT - 1)
        plsc.subcore_barrier()

        if k == 0:
            pltpu.sync_copy(deg_sh.at[pl.ds(row_base, _RT), :], degloc_v)

            @plsc.parallel_loop(0, _RT, 1, unroll=8)
            def _invd(r):
                degloc_v[r, :] = 1.0 / (degloc_v[r, :] + 1.0)

        def _fin(i, _):
            rows = pl.ds(row_base + i * _FR, _FR)
            pltpu.sync_copy(acc_sh.at[rows, :], facc_v)
            pltpu.sync_copy(tblf.at[rows, :], fh_v)

            @plsc.parallel_loop(0, _FR, 1, unroll=4)
            def _rowbody(r):
                iv = degloc_v[i * _FR + r, :]
                for c4 in range(4):
                    sl = pl.ds(c4 * 16, 16)
                    facc_v[r, sl] = (facc_v[r, sl] + fh_v[r, sl]) * iv
                for l in range(2):
                    e = lax.bitcast_convert_type(
                        facc_v[r, pl.ds(l * 32, 16)], jnp.int32)
                    o = lax.bitcast_convert_type(
                        facc_v[r, pl.ds(l * 32 + 16, 16)], jnp.int32)
                    ew = lax.shift_right_logical(e + rnd, 16)
                    ow = (o + rnd) & mhi
                    fbw_v[r, pl.ds(l * 16, 16)] = ew | ow
            pltpu.sync_copy(facc_v, hs_hbm.at[k, c, rows, :])
            pltpu.sync_copy(fbw_v, hsb_hbm.at[k, c, rows, :])
            return 0
        lax.fori_loop(0, _NFIN, _fin, 0)


@functools.cache
def _make_sc_powers():
    return pl.kernel(
        _sc_body,
        out_type=(
            jax.ShapeDtypeStruct((_NITER, _NC, _NP, _HALF), jnp.float32),
            jax.ShapeDtypeStruct((_NITER, _NC, _NP, _W32), jnp.int32),
        ),
        mesh=plsc.VectorSubcoreMesh(core_axis_name="c", subcore_axis_name="s"),
        compiler_params=pltpu.CompilerParams(use_tc_tiling_on_sc=False),
        scratch_types=[
            pltpu.VMEM((_CPT, _CHUNK), jnp.int32),
            pltpu.VMEM((4, _CHUNK), jnp.int32),
            pltpu.VMEM((4, _CHUNK), jnp.int32),
            pltpu.VMEM((4, _CHUNK, _W32), jnp.int32),
            pltpu.VMEM((2, _CHUNK, _HALF), jnp.float32),
            pltpu.VMEM((_CHUNK, _DEGW), jnp.float32),
            pltpu.VMEM((_RT, _DEGW), jnp.float32),
            pltpu.VMEM((_FR, _HALF), jnp.float32),
            pltpu.VMEM((_FR, _HALF), jnp.float32),
            pltpu.VMEM((_FR, _W32), jnp.int32),
            pltpu.VMEM_SHARED((_NP, _HALF), jnp.float32),
            pltpu.VMEM_SHARED((_NP, _DEGW), jnp.float32),
            pltpu.SemaphoreType.DMA((4,)),
            pltpu.SemaphoreType.DMA((2,)),
        ],
    )


def _mlp_body(hs_ref, Wc_ref, bc_ref, W1_ref, b1_ref, W2_ref, b2_ref, o_ref):
    def mm(k, c, w_ref, r0):
        return jnp.dot(hs_ref[k, c], w_ref[pl.ds(r0, _HALF), :],
                       preferred_element_type=jnp.float32)

    acc2 = bc_ref[...].astype(jnp.float32)
    for idx in range(8):
        k, c = 1 + idx // 2, idx % 2
        acc2 = acc2 + mm(k, c, Wc_ref, idx * _HALF)
    out2 = jnp.maximum(acc2, 0.0)

    accm = b1_ref[...].astype(jnp.float32)
    for idx in range(8):
        k, c = idx // 2, idx % 2
        accm = accm + mm(k, c, W1_ref, idx * _HALF)
    accm = accm + jnp.dot(out2, W1_ref[pl.ds(_H, _H), :],
                          preferred_element_type=jnp.float32)
    hm = jnp.maximum(accm, 0.0)

    o_ref[...] = jnp.dot(hm, W2_ref[...],
                         preferred_element_type=jnp.float32) + b2_ref[...]


@functools.partial(jax.jit, static_argnames=())
def _mlp_head(hs, W_conv, b_conv, W1, b1, W2, b2):
    return pl.pallas_call(
        _mlp_body,
        grid=(_GRID,),
        in_specs=[
            pl.BlockSpec((_NITER, _NC, _BR, _HALF), lambda i: (0, 0, i, 0)),
            pl.BlockSpec((_H, _H), lambda i: (0, 0)),
            pl.BlockSpec((1, _H), lambda i: (0, 0)),
            pl.BlockSpec((2 * _H, _HID), lambda i: (0, 0)),
            pl.BlockSpec((1, _HID), lambda i: (0, 0)),
            pl.BlockSpec((_HID, _OUT), lambda i: (0, 0)),
            pl.BlockSpec((1, _OUT), lambda i: (0, 0)),
        ],
        out_specs=pl.BlockSpec((_BR, _OUT), lambda i: (i, 0)),
        out_shape=jax.ShapeDtypeStruct((_N, _OUT), jnp.float32),
    )(hs, W_conv, b_conv, W1, b1, W2, b2)


def kernel(x, edge_index, W_conv, b_conv, W1, b1, W2, b2):
    x2 = x.reshape(_N, _NC, _HALF).transpose(1, 0, 2)
    x_p = jnp.concatenate(
        [x2, jnp.zeros((_NC, _NP - _N, _HALF), jnp.float32)], axis=1)

    x_pi = x_p[:, :, _PERM]
    xw = jax.lax.bitcast_convert_type(
        x_p.astype(jnp.bfloat16).reshape(_NC, _NP, _W32, 2), jnp.int32)

    src = edge_index[0].astype(jnp.int32)
    dst = edge_index[1].astype(jnp.int32)
    packed = src | (dst << 14)
    pad = jnp.full((_EPAD - _E,), _N | (_N << 14), jnp.int32)
    edges = jnp.concatenate([packed, pad]).reshape(_NS, _CPT, _CHUNK)

    hs, _ = _make_sc_powers()(x_pi, xw, edges)

    rows_p = np.concatenate([b * _HALF + _PERM for b in range(8)])
    Wc_p = W_conv[rows_p, :]
    W1_p = jnp.concatenate([W1[:_H][rows_p, :], W1[_H:]], axis=0)

    return _mlp_head(hs, Wc_p, b_conv.reshape(1, _H), W1_p,
                     b1.reshape(1, _HID), W2, b2.reshape(1, _OUT))
